# baseline (speedup 1.0000x reference)
import functools

import jax
import jax.numpy as jnp
from jax import lax
from jax.experimental import pallas as pl
from jax.experimental.pallas import tpu as pltpu

N_NODES = 10000
D = 128
D_EDGE = 16
E = 320000

F32 = jnp.float32


def _lrelu(x):
    return jnp.maximum(x, 0.2 * x)


def _k1a_body(res_ref, ops_ref, wself_ref, wres_ref, wop_ref,
              aself_ref, aop_ref, ares_ref,
              selfres_ref, opproj_ref, resproj_ref,
              sa_ref, sb_ref, sc_ref, sd_ref, zself_ref, stats_ref):
    res = res_ref[...]
    ops = ops_ref[...]
    self_res = jnp.dot(res, wself_ref[...], preferred_element_type=F32)
    op_proj = jnp.dot(ops, wop_ref[0:128, :], preferred_element_type=F32)
    res_proj = jnp.dot(res, wres_ref[...], preferred_element_type=F32)
    selfres_ref[...] = self_res
    opproj_ref[...] = op_proj
    resproj_ref[...] = res_proj

    a_self = aself_ref[...]
    zs_pre = jnp.dot(self_res, a_self[0:128, :] + a_self[128:256, :],
                     preferred_element_type=F32)
    z_self = _lrelu(zs_pre)
    zself_ref[...] = z_self

    sa = jnp.dot(self_res, aop_ref[0:128, :], preferred_element_type=F32)
    sb = jnp.dot(op_proj, aop_ref[128:256, :], preferred_element_type=F32)
    sc = jnp.dot(self_res, ares_ref[0:128, :], preferred_element_type=F32)
    sd = jnp.dot(res_proj, ares_ref[128:256, :], preferred_element_type=F32)
    sa_ref[...] = sa
    sb_ref[...] = sb
    sc_ref[...] = sc
    sd_ref[...] = sd

    stats = jnp.concatenate(
        [jnp.max(z_self).reshape(1, 1), jnp.max(sa).reshape(1, 1),
         jnp.max(sb).reshape(1, 1), jnp.max(sc).reshape(1, 1),
         jnp.max(sd).reshape(1, 1), jnp.zeros((1, 3), F32)], axis=1)
    stats_ref[...] = stats


def _run_k1a(resources, operations, W_self, W_res, W_op,
             att_self, att_op, att_res):
    n = jax.ShapeDtypeStruct
    outs = [n((N_NODES, D), F32), n((N_NODES, D), F32), n((N_NODES, D), F32),
            n((N_NODES, 1), F32), n((N_NODES, 1), F32), n((N_NODES, 1), F32),
            n((N_NODES, 1), F32), n((N_NODES, 1), F32), n((1, 8), F32)]
    return pl.pallas_call(_k1a_body, out_shape=outs)(
        resources, operations, W_self, W_res, W_op, att_self, att_op, att_res)


_K1B_BLK = 4000


def _k1b_body(attr_ref, wop_ref, aop_ref, stats_ref, t_ref, m_ref, mt_scr):
    i = pl.program_id(0)
    v = jnp.dot(wop_ref[128:144, :], aop_ref[128:256, :],
                preferred_element_type=F32)
    t = jnp.dot(attr_ref[...], v, preferred_element_type=F32)
    t_ref[...] = t
    tmax = jnp.max(t)

    @pl.when(i == 0)
    def _init():
        mt_scr[0] = tmax

    @pl.when(i > 0)
    def _acc():
        mt_scr[0] = jnp.maximum(mt_scr[0], tmax)

    @pl.when(i == pl.num_programs(0) - 1)
    def _final():
        st = stats_ref[...]
        mt = mt_scr[0]
        u_need = st[0, 1] + st[0, 2] + mt
        u_same = st[0, 3] + st[0, 4]
        m = jnp.maximum(st[0, 0],
                        jnp.maximum(_lrelu(u_need), _lrelu(u_same)))
        m_ref[...] = jnp.full((1, 1), m, F32)


def _run_k1b(need_edge_attr, W_op, att_op, stats):
    grid = (E // _K1B_BLK,)
    return pl.pallas_call(
        _k1b_body,
        grid=grid,
        in_specs=[
            pl.BlockSpec((_K1B_BLK, D_EDGE), lambda i: (i, 0)),
            pl.BlockSpec((144, 128), lambda i: (0, 0)),
            pl.BlockSpec((256, 1), lambda i: (0, 0)),
            pl.BlockSpec((1, 8), lambda i: (0, 0)),
        ],
        out_specs=[
            pl.BlockSpec((_K1B_BLK, 1), lambda i: (i, 0)),
            pl.BlockSpec((1, 1), lambda i: (0, 0)),
        ],
        out_shape=[jax.ShapeDtypeStruct((E, 1), F32),
                   jax.ShapeDtypeStruct((1, 1), F32)],
        scratch_shapes=[pltpu.SMEM((1,), F32)],
    )(need_edge_attr, W_op, att_op, stats)


_K5_BLK = 2000


def _k5_body(selfres_ref, zself_ref, acc_ref, attr_ref, wop_ref,
             spart_ref, m_ref, out_ref, invs_scr):
    i = pl.program_id(0)
    m = m_ref[0, 0]

    @pl.when(i == 0)
    def _s():
        w_self_all = jnp.exp(zself_ref[...] - m)
        s = jnp.sum(w_self_all) + jnp.sum(spart_ref[...])
        invs_scr[0] = 1.0 / s

    z_blk = zself_ref[pl.ds(i * _K5_BLK, _K5_BLK), :]
    w_self = jnp.exp(z_blk - m)
    acc = acc_ref[0] + acc_ref[1]
    attr_acc = attr_ref[0] + attr_ref[1]
    attr_term = jnp.dot(attr_acc, wop_ref[128:144, :],
                        preferred_element_type=F32)
    x = (w_self * selfres_ref[...] + acc + attr_term) * invs_scr[0]
    out_ref[...] = jnp.where(x > 0, x, jnp.exp(x) - 1.0)


def _run_k5(self_res, z_self, acc_parts, attr_parts, W_op, s_parts, m):
    grid = (N_NODES // _K5_BLK,)
    return pl.pallas_call(
        _k5_body,
        grid=grid,
        in_specs=[
            pl.BlockSpec((_K5_BLK, D), lambda i: (i, 0)),
            pl.BlockSpec((N_NODES, 1), lambda i: (0, 0)),
            pl.BlockSpec((2, _K5_BLK, D), lambda i: (0, i, 0)),
            pl.BlockSpec((2, _K5_BLK, D_EDGE), lambda i: (0, i, 0)),
            pl.BlockSpec((144, 128), lambda i: (0, 0)),
            pl.BlockSpec((32, 16), lambda i: (0, 0)),
            pl.BlockSpec((1, 1), lambda i: (0, 0)),
        ],
        out_specs=pl.BlockSpec((_K5_BLK, D), lambda i: (i, 0)),
        out_shape=jax.ShapeDtypeStruct((N_NODES, D), F32),
        scratch_shapes=[pltpu.SMEM((1,), F32)],
    )(self_res, z_self, acc_parts, attr_parts, W_op, s_parts, m)


def _edge_phase_jax(op_proj, res_proj, need_edge_index, need_edge_attr,
                    same_edge_index, t, sa, sb, sc, sd, m):
    z_need = _lrelu(sa[need_edge_index[1]] + sb[need_edge_index[0]] + t)
    w_need = jnp.exp(z_need - m)
    z_same = _lrelu(sc[same_edge_index[1]] + sd[same_edge_index[0]])
    w_same = jnp.exp(z_same - m)
    acc = jax.ops.segment_sum(w_need[:, None] * op_proj[need_edge_index[0]],
                              need_edge_index[1], num_segments=N_NODES)
    acc = acc + jax.ops.segment_sum(
        w_same[:, None] * res_proj[same_edge_index[0]],
        same_edge_index[1], num_segments=N_NODES)
    attr_acc = jax.ops.segment_sum(w_need[:, None] * need_edge_attr,
                                   need_edge_index[1], num_segments=N_NODES)
    acc_parts = jnp.stack([acc, jnp.zeros_like(acc)])
    attr_parts = jnp.stack([attr_acc, jnp.zeros_like(attr_acc)])
    s_parts = jnp.zeros((32, 16), F32).at[0, 0].set(
        jnp.sum(w_need) + jnp.sum(w_same))
    return acc_parts, attr_parts, s_parts


def kernel(resources, operations, need_edge_index, need_edge_attr,
           same_edge_index, W_self, W_res, W_op, att_self, att_op, att_res):
    (self_res, op_proj, res_proj, sa, sb, sc, sd, z_self,
     stats) = _run_k1a(resources, operations, W_self, W_res, W_op,
                       att_self, att_op, att_res)
    t, m = _run_k1b(need_edge_attr, W_op, att_op, stats)

    acc_parts, attr_parts, s_parts = _edge_phase_jax(
        op_proj, res_proj, need_edge_index, need_edge_attr, same_edge_index,
        t[:, 0], sa[:, 0], sb[:, 0], sc[:, 0], sd[:, 0], m[0, 0])

    return _run_k5(self_res, z_self, acc_parts, attr_parts, W_op, s_parts, m)


# speedup vs baseline: 14.7316x; 14.7316x over previous
import jax
import jax.numpy as jnp
from jax import lax
from jax.experimental import pallas as pl
from jax.experimental.pallas import tpu as pltpu
from jax.experimental.pallas import tpu_sc as plsc

N_NODES = 10000
D = 128
D_EDGE = 16
E = 320000

F32 = jnp.float32


def _lrelu(x):
    return jnp.maximum(x, 0.2 * x)


def _k1a_body(res_ref, ops_ref, wself_ref, wres_ref, wop_ref,
              aself_ref, aop_ref, ares_ref,
              selfres_ref, opproj_ref, resproj_ref,
              sa_ref, sb_ref, sc_ref, sd_ref, zself_ref, stats_ref):
    res = res_ref[...]
    ops = ops_ref[...]
    self_res = jnp.dot(res, wself_ref[...], preferred_element_type=F32)
    op_proj = jnp.dot(ops, wop_ref[0:128, :], preferred_element_type=F32)
    res_proj = jnp.dot(res, wres_ref[...], preferred_element_type=F32)
    selfres_ref[...] = self_res
    opproj_ref[...] = op_proj
    resproj_ref[...] = res_proj

    a_self = aself_ref[...]
    zs_pre = jnp.dot(self_res, a_self[0:128, :] + a_self[128:256, :],
                     preferred_element_type=F32)
    z_self = _lrelu(zs_pre)
    zself_ref[...] = z_self

    sa = jnp.dot(self_res, aop_ref[0:128, :], preferred_element_type=F32)
    sb = jnp.dot(op_proj, aop_ref[128:256, :], preferred_element_type=F32)
    sc = jnp.dot(self_res, ares_ref[0:128, :], preferred_element_type=F32)
    sd = jnp.dot(res_proj, ares_ref[128:256, :], preferred_element_type=F32)
    sa_ref[...] = sa
    sb_ref[...] = sb
    sc_ref[...] = sc
    sd_ref[...] = sd

    stats = jnp.concatenate(
        [jnp.max(z_self).reshape(1, 1), jnp.max(sa).reshape(1, 1),
         jnp.max(sb).reshape(1, 1), jnp.max(sc).reshape(1, 1),
         jnp.max(sd).reshape(1, 1), jnp.zeros((1, 3), F32)], axis=1)
    stats_ref[...] = stats


def _run_k1a(resources, operations, W_self, W_res, W_op,
             att_self, att_op, att_res):
    n = jax.ShapeDtypeStruct
    outs = [n((N_NODES, D), F32), n((N_NODES, D), F32), n((N_NODES, D), F32),
            n((N_NODES, 1), F32), n((N_NODES, 1), F32), n((N_NODES, 1), F32),
            n((N_NODES, 1), F32), n((N_NODES, 1), F32), n((1, 8), F32)]
    return pl.pallas_call(_k1a_body, out_shape=outs)(
        resources, operations, W_self, W_res, W_op, att_self, att_op, att_res)


_K1B_BLK = 12800


def _k1b_body(attrp_ref, wop_ref, aop_ref, stats_ref, t_ref, m_ref, mt_scr):
    i = pl.program_id(0)
    v = jnp.dot(wop_ref[128:144, :], aop_ref[128:256, :],
                preferred_element_type=F32)
    vrep = jnp.concatenate([v[:, 0]] * 8, axis=0)
    krow = lax.broadcasted_iota(jnp.int32, (128, 8), 0)
    jcol = lax.broadcasted_iota(jnp.int32, (128, 8), 1)
    v8 = jnp.where(krow // 16 == jcol, vrep[:, None], 0.0)
    t = jnp.dot(attrp_ref[...], v8, preferred_element_type=F32)
    t_ref[...] = t
    tmax = jnp.max(t)

    @pl.when(i == 0)
    def _init():
        mt_scr[0] = tmax

    @pl.when(i > 0)
    def _acc():
        mt_scr[0] = jnp.maximum(mt_scr[0], tmax)

    @pl.when(i == pl.num_programs(0) - 1)
    def _final():
        st = stats_ref[...]
        mt = mt_scr[0]
        u_need = st[0, 1] + st[0, 2] + mt
        u_same = st[0, 3] + st[0, 4]
        m = jnp.maximum(st[0, 0],
                        jnp.maximum(_lrelu(u_need), _lrelu(u_same)))
        m_ref[...] = jnp.full((1, 1), m, F32)


def _run_k1b(attr_packed, W_op, att_op, stats):
    grid = (E // _K1B_BLK,)
    return pl.pallas_call(
        _k1b_body,
        grid=grid,
        in_specs=[
            pl.BlockSpec((_K1B_BLK // 8, 128), lambda i: (i, 0)),
            pl.BlockSpec((144, 128), lambda i: (0, 0)),
            pl.BlockSpec((256, 1), lambda i: (0, 0)),
            pl.BlockSpec((1, 8), lambda i: (0, 0)),
        ],
        out_specs=[
            pl.BlockSpec((_K1B_BLK // 8, 8), lambda i: (i, 0)),
            pl.BlockSpec((1, 1), lambda i: (0, 0)),
        ],
        out_shape=[jax.ShapeDtypeStruct((E // 8, 8), F32),
                   jax.ShapeDtypeStruct((1, 1), F32)],
        scratch_shapes=[pltpu.SMEM((1,), F32)],
    )(attr_packed, W_op, att_op, stats)


_CH = 80
_EPW = 10000
_NCH = _EPW // _CH
_NPAD = 10240
_RPT = _NPAD // 16


def _sc_mesh():
    return plsc.VectorSubcoreMesh(core_axis_name="c", subcore_axis_name="s",
                                  num_cores=2, num_subcores=16)


def _k4_body(opproj_hbm, resproj_hbm, nsrc_hbm, ndst_hbm, ssrc_hbm, sdst_hbm,
             t_hbm, sa_hbm, sb_hbm, sc_hbm, sd_hbm, m_hbm,
             acc_out, wneed_out, ssum_out,
             acc_sp, ta_v, tb_v,
             srcA, dstA, tA, rowsA, srcB, dstB, tB, rowsB,
             wv, m_v, ssum_v, semAg, semBg, semAs, semBs):
    c = lax.axis_index("c")
    s = lax.axis_index("s")
    wid = c * 16 + s
    zeros16 = jnp.zeros((16,), F32)

    pltpu.sync_copy(m_hbm, m_v)
    m = m_v[pl.ds(0, 16)][0]

    def _z1(i, carry):
        rowsA[i // 8, pl.ds((i % 8) * 16, 16)] = zeros16
        return carry
    lax.fori_loop(0, _CH * 8, _z1, 0)

    for k in range(_RPT // _CH):
        pltpu.sync_copy(rowsA, acc_sp.at[pl.ds(s * _RPT + k * _CH, _CH)])
    plsc.subcore_barrier()

    def _set_loop(table_hbm, src_hbm, dst_hbm, has_t, ssum):
        def small_issue(k, srcX, dstX, tX, semX):
            base = wid * _EPW + k * _CH
            pltpu.async_copy(src_hbm.at[pl.ds(base, _CH)], srcX, semX)
            pltpu.async_copy(dst_hbm.at[pl.ds(base, _CH)], dstX, semX)
            if has_t:
                pltpu.async_copy(t_hbm.at[pl.ds(base, _CH)], tX, semX)

        def small_wait(srcX, dstX, tX, semX):
            pltpu.make_async_copy(src_hbm.at[pl.ds(0, _CH)], srcX,
                                  semX).wait()
            pltpu.make_async_copy(dst_hbm.at[pl.ds(0, _CH)], dstX,
                                  semX).wait()
            if has_t:
                pltpu.make_async_copy(t_hbm.at[pl.ds(0, _CH)], tX,
                                      semX).wait()

        def gather_issue(srcX, rowsX, semX):
            pltpu.async_copy(table_hbm.at[srcX], rowsX, semX)

        def gather_wait(rowsX, semX):
            pltpu.make_async_copy(table_hbm.at[pl.ds(0, _CH)], rowsX,
                                  semX).wait()

        def compute(k, srcX, dstX, tX, rowsX, ssum):
            def grp(g, ssum2):
                sl = pl.ds(g * 16, 16)
                u = (plsc.load_gather(ta_v, [dstX[sl]])
                     + plsc.load_gather(tb_v, [srcX[sl]]))
                if has_t:
                    u = u + tX[sl]
                z = jnp.maximum(u, 0.2 * u)
                w16 = jnp.exp(z - m)
                if has_t:
                    wv[sl] = w16
                for q in range(16):
                    wb = jnp.full((16,), w16[q], F32)
                    for j in range(8):
                        slj = pl.ds(j * 16, 16)
                        rowsX[g * 16 + q, slj] = rowsX[g * 16 + q, slj] * wb
                return ssum2 + w16
            ssum = lax.fori_loop(0, _CH // 16, grp, ssum)
            if has_t:
                base = wid * _EPW + k * _CH
                pltpu.sync_copy(wv, wneed_out.at[pl.ds(base, _CH)])
            pltpu.sync_copy(rowsX, acc_sp.at[dstX], add=True)
            return ssum

        small_issue(0, srcA, dstA, tA, semAs)
        small_wait(srcA, dstA, tA, semAs)
        gather_issue(srcA, rowsA, semAg)
        small_issue(1, srcB, dstB, tB, semBs)

        def body(j, ssum):
            kA = 2 * j
            kB = 2 * j + 1
            small_wait(srcB, dstB, tB, semBs)
            gather_issue(srcB, rowsB, semBg)
            gather_wait(rowsA, semAg)
            ssum = compute(kA, srcA, dstA, tA, rowsA, ssum)
            small_issue(kA + 2, srcA, dstA, tA, semAs)
            small_wait(srcA, dstA, tA, semAs)
            gather_issue(srcA, rowsA, semAg)
            gather_wait(rowsB, semBg)
            ssum = compute(kB, srcB, dstB, tB, rowsB, ssum)

            @pl.when(j < _NCH // 2 - 1)
            def _more():
                small_issue(kB + 2, srcB, dstB, tB, semBs)
            return ssum
        ssum = lax.fori_loop(0, _NCH // 2, body, ssum)

        gather_wait(rowsA, semAg)
        ssum = compute(_NCH - 1, srcA, dstA, tA, rowsA, ssum)
        return ssum

    pltpu.sync_copy(sa_hbm, ta_v)
    pltpu.sync_copy(sb_hbm, tb_v)
    ssum = _set_loop(opproj_hbm, nsrc_hbm, ndst_hbm, True, zeros16)
    pltpu.sync_copy(sc_hbm, ta_v)
    pltpu.sync_copy(sd_hbm, tb_v)
    ssum = _set_loop(resproj_hbm, ssrc_hbm, sdst_hbm, False, ssum)
    ssum_v[...] = ssum

    plsc.subcore_barrier()
    row0 = s * _RPT
    out0 = c * _NPAD + row0
    pltpu.sync_copy(acc_sp.at[pl.ds(row0, _RPT)], acc_out.at[pl.ds(out0, _RPT)])
    pltpu.sync_copy(ssum_v, ssum_out.at[pl.ds(wid * 16, 16)])


def _run_k4(op_proj, res_proj, need_src, need_dst, same_src, same_dst,
            t, sa, sb, sc, sd, m16):
    n = jax.ShapeDtypeStruct
    f = pl.kernel(
        _k4_body,
        out_type=[n((2 * _NPAD, D), F32), n((E,), F32), n((512,), F32)],
        mesh=_sc_mesh(),
        scratch_types=[
            pltpu.VMEM_SHARED((_NPAD, D), F32),
            pltpu.VMEM((N_NODES,), F32),
            pltpu.VMEM((N_NODES,), F32),
            pltpu.VMEM((_CH,), jnp.int32),
            pltpu.VMEM((_CH,), jnp.int32),
            pltpu.VMEM((_CH,), F32),
            pltpu.VMEM((_CH, D), F32),
            pltpu.VMEM((_CH,), jnp.int32),
            pltpu.VMEM((_CH,), jnp.int32),
            pltpu.VMEM((_CH,), F32),
            pltpu.VMEM((_CH, D), F32),
            pltpu.VMEM((_CH,), F32),
            pltpu.VMEM((16,), F32),
            pltpu.VMEM((16,), F32),
            pltpu.SemaphoreType.DMA,
            pltpu.SemaphoreType.DMA,
            pltpu.SemaphoreType.DMA,
            pltpu.SemaphoreType.DMA,
        ],
        compiler_params=pltpu.CompilerParams(
            needs_layout_passes=False,
            internal_scratch_in_bytes=16384,
        ),
    )
    acc, w_need, ssum = f(op_proj, res_proj, need_src, need_dst,
                          same_src, same_dst, t, sa, sb, sc, sd, m16)
    return acc.reshape(2, _NPAD, D)[:, :N_NODES], w_need, ssum


_ACH = 128
_NCHB = E // _ACH


def _k4b_body(attrp_hbm, ndst_hbm, wneed_hbm, attr_out,
              attr_sp, dstA, wA, packA, rowsA, dstB, wB, packB, rowsB,
              semAg, semBg, semAs, semBs):
    c = lax.axis_index("c")
    s = lax.axis_index("s")
    wid = c * 16 + s
    zeros16 = jnp.zeros((16,), F32)

    def _zero_rows(rowsX):
        def _z1(i, carry):
            rowsX[i // 8, pl.ds((i % 8) * 16, 16)] = zeros16
            return carry
        lax.fori_loop(0, _ACH * 8, _z1, 0)
    _zero_rows(rowsA)
    _zero_rows(rowsB)

    for k in range(_RPT // _ACH):
        pltpu.sync_copy(rowsA, attr_sp.at[pl.ds(s * _RPT + k * _ACH, _ACH)])
    plsc.subcore_barrier()

    def small_issue(k, dstX, wX, packX, semX):
        base = k * _ACH
        pltpu.async_copy(ndst_hbm.at[pl.ds(base, _ACH)], dstX, semX)
        pltpu.async_copy(wneed_hbm.at[pl.ds(base, _ACH)], wX, semX)
        pltpu.async_copy(attrp_hbm.at[pl.ds(k * 16, 16)], packX, semX)

    def small_wait(dstX, wX, packX, semX):
        pltpu.make_async_copy(ndst_hbm.at[pl.ds(0, _ACH)], dstX, semX).wait()
        pltpu.make_async_copy(wneed_hbm.at[pl.ds(0, _ACH)], wX, semX).wait()
        pltpu.make_async_copy(attrp_hbm.at[pl.ds(0, 16)], packX, semX).wait()

    def compute(dstX, wX, packX, rowsX):
        for g in range(_ACH // 16):
            wg = wX[pl.ds(g * 16, 16)]
            for q in range(16):
                e = g * 16 + q
                wb = jnp.full((16,), wg[q], F32)
                a = packX[e // 8, pl.ds((e % 8) * 16, 16)]
                rowsX[e, pl.ds(0, 16)] = a * wb
        pltpu.sync_copy(rowsX, attr_sp.at[dstX], add=True)

    def valid(j):
        return wid + j * 32 < _NCHB

    def chunk_of(j):
        return wid + j * 32

    _NJP = 39

    small_issue(chunk_of(0), dstA, wA, packA, semAs)
    small_issue(chunk_of(1), dstB, wB, packB, semBs)

    def body(i, carry):
        jA = 2 * i
        jB = 2 * i + 1
        small_wait(dstA, wA, packA, semAs)
        compute(dstA, wA, packA, rowsA)

        @pl.when(valid(jA + 2))
        def _nextA():
            small_issue(chunk_of(jA + 2), dstA, wA, packA, semAs)
        small_wait(dstB, wB, packB, semBs)
        compute(dstB, wB, packB, rowsB)

        @pl.when(valid(jB + 2))
        def _nextB():
            small_issue(chunk_of(jB + 2), dstB, wB, packB, semBs)
        return carry
    lax.fori_loop(0, _NJP, body, 0)

    @pl.when(valid(78))
    def _tail():
        small_wait(dstA, wA, packA, semAs)
        compute(dstA, wA, packA, rowsA)

    plsc.subcore_barrier()
    row0 = s * _RPT
    out0 = c * _NPAD + row0
    pltpu.sync_copy(attr_sp.at[pl.ds(row0, _RPT)],
                    attr_out.at[pl.ds(out0, _RPT)])


def _run_k4b(attr_packed, need_dst, w_need):
    n = jax.ShapeDtypeStruct
    f = pl.kernel(
        _k4b_body,
        out_type=n((2 * _NPAD, D), F32),
        mesh=_sc_mesh(),
        scratch_types=[
            pltpu.VMEM_SHARED((_NPAD, D), F32),
            pltpu.VMEM((_ACH,), jnp.int32),
            pltpu.VMEM((_ACH,), F32),
            pltpu.VMEM((16, 128), F32),
            pltpu.VMEM((_ACH, D), F32),
            pltpu.VMEM((_ACH,), jnp.int32),
            pltpu.VMEM((_ACH,), F32),
            pltpu.VMEM((16, 128), F32),
            pltpu.VMEM((_ACH, D), F32),
            pltpu.SemaphoreType.DMA,
            pltpu.SemaphoreType.DMA,
            pltpu.SemaphoreType.DMA,
            pltpu.SemaphoreType.DMA,
        ],
        compiler_params=pltpu.CompilerParams(
            needs_layout_passes=False,
            internal_scratch_in_bytes=16384,
        ),
    )
    attr_acc = f(attr_packed, need_dst, w_need)
    return attr_acc.reshape(2, _NPAD, D)[:, :N_NODES, :D_EDGE]


_K5_BLK = 2000


def _k5_body(selfres_ref, zself_ref, acc_ref, attr_ref, wop_ref,
             spart_ref, m_ref, out_ref, invs_scr):
    i = pl.program_id(0)
    m = m_ref[0, 0]

    @pl.when(i == 0)
    def _s():
        w_self_all = jnp.exp(zself_ref[...] - m)
        s = jnp.sum(w_self_all) + jnp.sum(spart_ref[...])
        invs_scr[0] = 1.0 / s

    z_blk = zself_ref[pl.ds(i * _K5_BLK, _K5_BLK), :]
    w_self = jnp.exp(z_blk - m)
    acc = acc_ref[0] + acc_ref[1]
    attr_acc = attr_ref[0] + attr_ref[1]
    attr_term = jnp.dot(attr_acc, wop_ref[128:144, :],
                        preferred_element_type=F32)
    x = (w_self * selfres_ref[...] + acc + attr_term) * invs_scr[0]
    out_ref[...] = jnp.where(x > 0, x, jnp.exp(x) - 1.0)


def _run_k5(self_res, z_self, acc_parts, attr_parts, W_op, s_parts, m):
    grid = (N_NODES // _K5_BLK,)
    return pl.pallas_call(
        _k5_body,
        grid=grid,
        in_specs=[
            pl.BlockSpec((_K5_BLK, D), lambda i: (i, 0)),
            pl.BlockSpec((N_NODES, 1), lambda i: (0, 0)),
            pl.BlockSpec((2, _K5_BLK, D), lambda i: (0, i, 0)),
            pl.BlockSpec((2, _K5_BLK, D_EDGE), lambda i: (0, i, 0)),
            pl.BlockSpec((144, 128), lambda i: (0, 0)),
            pl.BlockSpec((32, 16), lambda i: (0, 0)),
            pl.BlockSpec((1, 1), lambda i: (0, 0)),
        ],
        out_specs=pl.BlockSpec((_K5_BLK, D), lambda i: (i, 0)),
        out_shape=jax.ShapeDtypeStruct((N_NODES, D), F32),
        scratch_shapes=[pltpu.SMEM((1,), F32)],
    )(self_res, z_self, acc_parts, attr_parts, W_op, s_parts, m)


def kernel(resources, operations, need_edge_index, need_edge_attr,
           same_edge_index, W_self, W_res, W_op, att_self, att_op, att_res):
    (self_res, op_proj, res_proj, sa, sb, sc, sd, z_self,
     stats) = _run_k1a(resources, operations, W_self, W_res, W_op,
                       att_self, att_op, att_res)
    attr_packed = need_edge_attr.reshape(E // 8, 128)
    t, m = _run_k1b(attr_packed, W_op, att_op, stats)

    m16 = jnp.broadcast_to(m.reshape(1), (16,))
    acc_parts, w_need, s_parts = _run_k4(
        op_proj, res_proj,
        need_edge_index[0], need_edge_index[1],
        same_edge_index[0], same_edge_index[1],
        t.reshape(E), sa.reshape(N_NODES), sb.reshape(N_NODES),
        sc.reshape(N_NODES), sd.reshape(N_NODES), m16)
    attr_parts = _run_k4b(attr_packed, need_edge_index[1], w_need)

    return _run_k5(self_res, z_self, acc_parts, attr_parts, W_op,
                   s_parts.reshape(32, 16), m)
